# Initial kernel scaffold; baseline (speedup 1.0000x reference)
import jax
import jax.numpy as jnp
from jax.experimental import pallas as pl


def kernel(x, edge_index, W1, b1, W2, b2):
    raise NotImplementedError("write your pallas kernel here")


# speedup vs baseline: 7.4596x; 7.4596x over previous
import functools

import jax
import jax.numpy as jnp
from jax import lax
from jax.experimental import pallas as pl
from jax.experimental.pallas import tpu as pltpu
from jax.experimental.pallas import tpu_sc as plsc

N = 100000
E = 1600000
IN_DIM, HID_DIM, OUT_DIM = 32, 64, 128

NPAD = 100096
RPT = NPAD // 16
ZR = RPT // 8
K = 128
EPAD = 1601536
EHALF = EPAD // 2

RB = 256
NBLK = NPAD // RB

_MESH = plsc.VectorSubcoreMesh(core_axis_name="c", subcore_axis_name="s")


def _fill(ref, rows, value):
    def body(i, carry):
        ref[i] = jnp.full((16,), value, jnp.float32)
        return carry
    lax.fori_loop(0, rows, body, 0)


_ZCH = RPT // K + 1


def _zero_acc(acc, zeros, idxv, s):
    def chunk(i, carry):
        base = jnp.minimum(s * RPT + i * K, NPAD - K)

        def zidx(j, carry2):
            idxv[pl.ds(j * 16, 16)] = lax.iota(jnp.int32, 16) + (base + j * 16)
            return carry2

        lax.fori_loop(0, K // 16, zidx, 0)
        pltpu.sync_copy(zeros, acc.at[idxv])
        return carry

    lax.fori_loop(0, _ZCH, chunk, 0)


@functools.partial(
    pl.kernel,
    out_type=jax.ShapeDtypeStruct((2 * NPAD, 16), jnp.float32),
    mesh=_MESH,
    compiler_params=pltpu.CompilerParams(use_tc_tiling_on_sc=False),
    scratch_types=[
        pltpu.VMEM((K, 16), jnp.float32),
        pltpu.VMEM((K,), jnp.int32),
        pltpu.VMEM_SHARED((NPAD, 16), jnp.float32),
    ],
)
def _deg_kernel(dst_hbm, out_hbm, ones, dstv, acc):
    c = lax.axis_index("c")
    s = lax.axis_index("s")
    _fill(ones, K, 0.0)
    _zero_acc(acc, ones, dstv, s)
    plsc.subcore_barrier()
    _fill(ones, K, 1.0)
    cpt = EHALF // 16 // K

    def body(i, carry):
        base = c * EHALF + (s * cpt + i) * K
        pltpu.sync_copy(dst_hbm.at[pl.ds(base, K)], dstv)
        pltpu.sync_copy(ones, acc.at[dstv], add=True)
        return carry

    lax.fori_loop(0, cpt, body, 0)
    plsc.subcore_barrier()
    pltpu.sync_copy(acc.at[pl.ds(s * RPT, RPT)],
                    out_hbm.at[pl.ds(c * NPAD + s * RPT, RPT)])


def _make_prop(G):
    G2 = G // 2
    cpt = EPAD // K // 16

    @functools.partial(
        pl.kernel,
        out_type=jax.ShapeDtypeStruct((G * NPAD, 16), jnp.float32),
        mesh=_MESH,
        compiler_params=pltpu.CompilerParams(use_tc_tiling_on_sc=False),
        scratch_types=[
            pltpu.VMEM((K,), jnp.int32),
            pltpu.VMEM((K,), jnp.int32),
            pltpu.VMEM((K,), jnp.int32),
            pltpu.VMEM((K, 16), jnp.float32),
            pltpu.VMEM_SHARED((NPAD, 16), jnp.float32),
            pltpu.SemaphoreType.DMA,
        ],
    )
    def prop(table, src_hbm, dst_hbm, out_hbm,
             srcv, gidxv, dstv, rows, acc, sem):
        c = lax.axis_index("c")
        s = lax.axis_index("s")
        for gi in range(G2):
            g = c * G2 + gi
            _fill(rows, K, 0.0)
            _zero_acc(acc, rows, dstv, s)
            plsc.subcore_barrier()

            def body(i, carry):
                base = (s * cpt + i) * K
                pltpu.sync_copy(src_hbm.at[pl.ds(base, K)], srcv)
                pltpu.sync_copy(dst_hbm.at[pl.ds(base, K)], dstv)

                def off(j2, c2):
                    gidxv[pl.ds(j2 * 16, 16)] = srcv[pl.ds(j2 * 16, 16)] * G + g
                    return c2

                lax.fori_loop(0, K // 16, off, 0)
                pltpu.async_copy(table.at[gidxv], rows, sem).wait()
                pltpu.sync_copy(rows, acc.at[dstv], add=True)
                return carry

            lax.fori_loop(0, cpt, body, 0)
            plsc.subcore_barrier()
            pltpu.sync_copy(acc.at[pl.ds(s * RPT, RPT)],
                            out_hbm.at[pl.ds(g * NPAD + s * RPT, RPT)])
    return prop


_prop_g2 = _make_prop(2)
_prop_g4 = _make_prop(4)


def _tc_prep(deg16, x):
    def body(deg_ref, x_ref, tab_ref, dis_ref, inv_ref):
        deg = deg_ref[0, :, 0:1] + deg_ref[1, :, 0:1] + 1.0
        d = lax.rsqrt(deg)
        tab_ref[...] = x_ref[...] * d
        dis_ref[...] = d
        inv_ref[...] = 1.0 / deg

    return pl.pallas_call(
        body,
        grid=(NBLK,),
        in_specs=[
            pl.BlockSpec((2, RB, 16), lambda i: (0, i, 0)),
            pl.BlockSpec((RB, IN_DIM), lambda i: (i, 0)),
        ],
        out_specs=[
            pl.BlockSpec((RB, IN_DIM), lambda i: (i, 0)),
            pl.BlockSpec((RB, 1), lambda i: (i, 0)),
            pl.BlockSpec((RB, 1), lambda i: (i, 0)),
        ],
        out_shape=[
            jax.ShapeDtypeStruct((NPAD, IN_DIM), jnp.float32),
            jax.ShapeDtypeStruct((NPAD, 1), jnp.float32),
            jax.ShapeDtypeStruct((NPAD, 1), jnp.float32),
        ],
    )(deg16, x)


def _tc_layer1(acc1, x, dis, inv, W1, b1):
    def body(acc_ref, x_ref, dis_ref, inv_ref, w_ref, b_ref, tab_ref, hsl_ref):
        accc = jnp.concatenate([acc_ref[0], acc_ref[1]], axis=1)
        d = dis_ref[...]
        iv = inv_ref[...]
        p1 = d * accc + iv * x_ref[...]
        h = jnp.dot(p1, w_ref[...], preferred_element_type=jnp.float32)
        h = jnp.maximum(h + b_ref[...], 0.0)
        tab_ref[...] = h * d
        hsl_ref[...] = h * iv

    return pl.pallas_call(
        body,
        grid=(NBLK,),
        in_specs=[
            pl.BlockSpec((2, RB, 16), lambda i: (0, i, 0)),
            pl.BlockSpec((RB, IN_DIM), lambda i: (i, 0)),
            pl.BlockSpec((RB, 1), lambda i: (i, 0)),
            pl.BlockSpec((RB, 1), lambda i: (i, 0)),
            pl.BlockSpec((IN_DIM, HID_DIM), lambda i: (0, 0)),
            pl.BlockSpec((1, HID_DIM), lambda i: (0, 0)),
        ],
        out_specs=[
            pl.BlockSpec((RB, HID_DIM), lambda i: (i, 0)),
            pl.BlockSpec((RB, HID_DIM), lambda i: (i, 0)),
        ],
        out_shape=[
            jax.ShapeDtypeStruct((NPAD, HID_DIM), jnp.float32),
            jax.ShapeDtypeStruct((NPAD, HID_DIM), jnp.float32),
        ],
    )(acc1, x, dis, inv, W1, b1)


def _tc_layer2(acc2, hsl, dis, W2, b2):
    def body(acc_ref, hsl_ref, dis_ref, w_ref, b_ref, out_ref):
        accc = jnp.concatenate(
            [acc_ref[0], acc_ref[1], acc_ref[2], acc_ref[3]], axis=1)
        p2 = dis_ref[...] * accc + hsl_ref[...]
        o = jnp.dot(p2, w_ref[...], preferred_element_type=jnp.float32)
        out_ref[...] = o + b_ref[...]

    return pl.pallas_call(
        body,
        grid=(NBLK,),
        in_specs=[
            pl.BlockSpec((4, RB, 16), lambda i: (0, i, 0)),
            pl.BlockSpec((RB, HID_DIM), lambda i: (i, 0)),
            pl.BlockSpec((RB, 1), lambda i: (i, 0)),
            pl.BlockSpec((HID_DIM, OUT_DIM), lambda i: (0, 0)),
            pl.BlockSpec((1, OUT_DIM), lambda i: (0, 0)),
        ],
        out_specs=pl.BlockSpec((RB, OUT_DIM), lambda i: (i, 0)),
        out_shape=jax.ShapeDtypeStruct((N, OUT_DIM), jnp.float32),
    )(acc2, hsl, dis, W2, b2)


@functools.partial(
    pl.kernel,
    out_type=jax.ShapeDtypeStruct((NPAD, IN_DIM), jnp.float32),
    mesh=_MESH,
    compiler_params=pltpu.CompilerParams(use_tc_tiling_on_sc=False),
    scratch_types=[
        pltpu.VMEM((NPAD // 32,), jnp.int32),
        pltpu.VMEM((NPAD // 32, IN_DIM), jnp.float32),
        pltpu.SemaphoreType.DMA,
    ],
)
def _skel_gather(table_hbm, idx_hbm, out_hbm, idx_v, rows_v, sem):
    bpw = NPAD // 32
    wid = lax.axis_index("s") * 2 + lax.axis_index("c")
    base = wid * bpw
    pltpu.sync_copy(idx_hbm.at[pl.ds(base, bpw)], idx_v)
    pltpu.async_copy(table_hbm.at[idx_v], rows_v, sem).wait()
    pltpu.sync_copy(rows_v, out_hbm.at[pl.ds(base, bpw)])


def kernel(x, edge_index, W1, b1, W2, b2):
    src = edge_index[0].astype(jnp.int32)
    dst = edge_index[1].astype(jnp.int32)
    pad = jnp.full((EPAD - E,), N, jnp.int32)
    src_p = jnp.concatenate([src, pad])
    dst_p = jnp.concatenate([dst, pad])

    deg16 = _deg_kernel(dst_p).reshape(2, NPAD, 16)
    tab1, dis, inv = _tc_prep(deg16, x)
    acc1 = _prop_g2(tab1.reshape(2 * NPAD, 16), src_p, dst_p)
    tab2, hsl = _tc_layer1(acc1.reshape(2, NPAD, 16), x, dis, inv,
                           W1, b1.reshape(1, HID_DIM))
    acc2 = _prop_g4(tab2.reshape(4 * NPAD, 16), src_p, dst_p)
    return _tc_layer2(acc2.reshape(4, NPAD, 16), hsl, dis,
                      W2, b2.reshape(1, OUT_DIM))


# speedup vs baseline: 14.7490x; 1.9772x over previous
import functools

import jax
import jax.numpy as jnp
from jax import lax
from jax.experimental import pallas as pl
from jax.experimental.pallas import tpu as pltpu
from jax.experimental.pallas import tpu_sc as plsc

N = 100000
E = 1600000
IN_DIM, HID_DIM, OUT_DIM = 32, 64, 128

NPAD = 100096
RPT = NPAD // 16
K = 128
EPAD = 1622016
EHALF = EPAD // 2

RB = 256
NBLK = NPAD // RB

_MESH = plsc.VectorSubcoreMesh(core_axis_name="c", subcore_axis_name="s")


def _fill(ref, rows, value):
    def body(i, carry):
        ref[i] = jnp.full((16,), value, jnp.float32)
        return carry
    lax.fori_loop(0, rows, body, 0)


_ZCH = RPT // K + 1


def _zero_acc(acc, zeros, idxv, s):
    def chunk(i, carry):
        base = jnp.minimum(s * RPT + i * K, NPAD - K)

        def zidx(j, carry2):
            idxv[pl.ds(j * 16, 16)] = lax.iota(jnp.int32, 16) + (base + j * 16)
            return carry2

        lax.fori_loop(0, K // 16, zidx, 0)
        pltpu.sync_copy(zeros, acc.at[idxv])
        return carry

    lax.fori_loop(0, _ZCH, chunk, 0)


SUP_D = 12
NSUP_D = 33
SUP_P = 12
NSUP_P = 66


@functools.partial(
    pl.kernel,
    out_type=jax.ShapeDtypeStruct((2 * NPAD, 16), jnp.float32),
    mesh=_MESH,
    compiler_params=pltpu.CompilerParams(use_tc_tiling_on_sc=False),
    scratch_types=[
        pltpu.VMEM((K, 16), jnp.float32),
        pltpu.VMEM((SUP_D, K), jnp.int32),
        pltpu.VMEM_SHARED((NPAD, 16), jnp.float32),
    ],
)
def _deg_kernel(dst2d_hbm, out_hbm, ones, dstb, acc):
    c = lax.axis_index("c")
    s = lax.axis_index("s")
    _fill(ones, K, 0.0)
    _zero_acc(acc, ones, dstb.at[0], s)
    plsc.subcore_barrier()
    _fill(ones, K, 1.0)
    cpt = EHALF // 16 // K

    def body(t, carry):
        rowbase = c * (EHALF // K) + s * cpt + t * SUP_D
        pltpu.sync_copy(dst2d_hbm.at[pl.ds(rowbase, SUP_D)], dstb)
        for u in range(SUP_D):
            pltpu.sync_copy(ones, acc.at[dstb.at[u]], add=True)
        return carry

    lax.fori_loop(0, NSUP_D, body, 0)
    plsc.subcore_barrier()
    pltpu.sync_copy(acc.at[pl.ds(s * RPT, RPT)],
                    out_hbm.at[pl.ds(c * NPAD + s * RPT, RPT)])


def _make_prop(G):
    G2 = G // 2
    cpt = EPAD // K // 16

    @functools.partial(
        pl.kernel,
        out_type=jax.ShapeDtypeStruct((G * NPAD, 16), jnp.float32),
        mesh=_MESH,
        compiler_params=pltpu.CompilerParams(use_tc_tiling_on_sc=False),
        scratch_types=[
            pltpu.VMEM((SUP_P, K), jnp.int32),
            pltpu.VMEM((SUP_P, K), jnp.int32),
            pltpu.VMEM((SUP_P, K, 16), jnp.float32),
            pltpu.VMEM_SHARED((NPAD, 16), jnp.float32),
            pltpu.SemaphoreType.DMA,
        ],
    )
    def prop(table, src2d_hbm, dst2d_hbm, out_hbm,
             gidxb, dstb, rows, acc, sem):
        c = lax.axis_index("c")
        s = lax.axis_index("s")
        for gi in range(G2):
            g = c * G2 + gi
            _fill(rows.at[0], K, 0.0)
            _zero_acc(acc, rows.at[0], dstb.at[0], s)
            plsc.subcore_barrier()

            def body(t, carry):
                rowbase = s * cpt + t * SUP_P
                pltpu.sync_copy(src2d_hbm.at[pl.ds(rowbase, SUP_P)], gidxb)
                pltpu.sync_copy(dst2d_hbm.at[pl.ds(rowbase, SUP_P)], dstb)

                def off(u, cu):
                    def off16(j2, c2):
                        gidxb[u, pl.ds(j2 * 16, 16)] = (
                            gidxb[u, pl.ds(j2 * 16, 16)] * G + g)
                        return c2
                    lax.fori_loop(0, K // 16, off16, 0)
                    return cu

                lax.fori_loop(0, SUP_P, off, 0)

                def fire(u, cu):
                    pltpu.async_copy(table.at[gidxb.at[u]], rows.at[u], sem)
                    return cu

                lax.fori_loop(0, SUP_P, fire, 0)

                def drain(u, cu):
                    pltpu.make_async_copy(
                        table.at[gidxb.at[u]], rows.at[u], sem).wait()
                    return cu

                lax.fori_loop(0, SUP_P, drain, 0)

                def scat(u, cu):
                    pltpu.sync_copy(rows.at[u], acc.at[dstb.at[u]], add=True)
                    return cu

                lax.fori_loop(0, SUP_P, scat, 0)
                return carry

            lax.fori_loop(0, NSUP_P, body, 0)
            plsc.subcore_barrier()
            pltpu.sync_copy(acc.at[pl.ds(s * RPT, RPT)],
                            out_hbm.at[pl.ds(g * NPAD + s * RPT, RPT)])
    return prop


_prop_g2 = _make_prop(2)
_prop_g4 = _make_prop(4)


def _tc_prep(deg16, x):
    def body(deg_ref, x_ref, tab_ref, dis_ref, inv_ref):
        deg = deg_ref[0, :, 0:1] + deg_ref[1, :, 0:1] + 1.0
        d = lax.rsqrt(deg)
        tab_ref[...] = x_ref[...] * d
        dis_ref[...] = d
        inv_ref[...] = 1.0 / deg

    return pl.pallas_call(
        body,
        grid=(NBLK,),
        in_specs=[
            pl.BlockSpec((2, RB, 16), lambda i: (0, i, 0)),
            pl.BlockSpec((RB, IN_DIM), lambda i: (i, 0)),
        ],
        out_specs=[
            pl.BlockSpec((RB, IN_DIM), lambda i: (i, 0)),
            pl.BlockSpec((RB, 1), lambda i: (i, 0)),
            pl.BlockSpec((RB, 1), lambda i: (i, 0)),
        ],
        out_shape=[
            jax.ShapeDtypeStruct((NPAD, IN_DIM), jnp.float32),
            jax.ShapeDtypeStruct((NPAD, 1), jnp.float32),
            jax.ShapeDtypeStruct((NPAD, 1), jnp.float32),
        ],
    )(deg16, x)


def _tc_layer1(acc1, x, dis, inv, W1, b1):
    def body(acc_ref, x_ref, dis_ref, inv_ref, w_ref, b_ref, tab_ref, hsl_ref):
        accc = jnp.concatenate([acc_ref[0], acc_ref[1]], axis=1)
        d = dis_ref[...]
        iv = inv_ref[...]
        p1 = d * accc + iv * x_ref[...]
        h = jnp.dot(p1, w_ref[...], preferred_element_type=jnp.float32)
        h = jnp.maximum(h + b_ref[...], 0.0)
        tab_ref[...] = h * d
        hsl_ref[...] = h * iv

    return pl.pallas_call(
        body,
        grid=(NBLK,),
        in_specs=[
            pl.BlockSpec((2, RB, 16), lambda i: (0, i, 0)),
            pl.BlockSpec((RB, IN_DIM), lambda i: (i, 0)),
            pl.BlockSpec((RB, 1), lambda i: (i, 0)),
            pl.BlockSpec((RB, 1), lambda i: (i, 0)),
            pl.BlockSpec((IN_DIM, HID_DIM), lambda i: (0, 0)),
            pl.BlockSpec((1, HID_DIM), lambda i: (0, 0)),
        ],
        out_specs=[
            pl.BlockSpec((RB, HID_DIM), lambda i: (i, 0)),
            pl.BlockSpec((RB, HID_DIM), lambda i: (i, 0)),
        ],
        out_shape=[
            jax.ShapeDtypeStruct((NPAD, HID_DIM), jnp.float32),
            jax.ShapeDtypeStruct((NPAD, HID_DIM), jnp.float32),
        ],
    )(acc1, x, dis, inv, W1, b1)


def _tc_layer2(acc2, hsl, dis, W2, b2):
    def body(acc_ref, hsl_ref, dis_ref, w_ref, b_ref, out_ref):
        accc = jnp.concatenate(
            [acc_ref[0], acc_ref[1], acc_ref[2], acc_ref[3]], axis=1)
        p2 = dis_ref[...] * accc + hsl_ref[...]
        o = jnp.dot(p2, w_ref[...], preferred_element_type=jnp.float32)
        out_ref[...] = o + b_ref[...]

    return pl.pallas_call(
        body,
        grid=(NBLK,),
        in_specs=[
            pl.BlockSpec((4, RB, 16), lambda i: (0, i, 0)),
            pl.BlockSpec((RB, HID_DIM), lambda i: (i, 0)),
            pl.BlockSpec((RB, 1), lambda i: (i, 0)),
            pl.BlockSpec((HID_DIM, OUT_DIM), lambda i: (0, 0)),
            pl.BlockSpec((1, OUT_DIM), lambda i: (0, 0)),
        ],
        out_specs=pl.BlockSpec((RB, OUT_DIM), lambda i: (i, 0)),
        out_shape=jax.ShapeDtypeStruct((N, OUT_DIM), jnp.float32),
    )(acc2, hsl, dis, W2, b2)


@functools.partial(
    pl.kernel,
    out_type=jax.ShapeDtypeStruct((NPAD, IN_DIM), jnp.float32),
    mesh=_MESH,
    compiler_params=pltpu.CompilerParams(use_tc_tiling_on_sc=False),
    scratch_types=[
        pltpu.VMEM((NPAD // 32,), jnp.int32),
        pltpu.VMEM((NPAD // 32, IN_DIM), jnp.float32),
        pltpu.SemaphoreType.DMA,
    ],
)
def _skel_gather(table_hbm, idx_hbm, out_hbm, idx_v, rows_v, sem):
    bpw = NPAD // 32
    wid = lax.axis_index("s") * 2 + lax.axis_index("c")
    base = wid * bpw
    pltpu.sync_copy(idx_hbm.at[pl.ds(base, bpw)], idx_v)
    pltpu.async_copy(table_hbm.at[idx_v], rows_v, sem).wait()
    pltpu.sync_copy(rows_v, out_hbm.at[pl.ds(base, bpw)])


def kernel(x, edge_index, W1, b1, W2, b2):
    src = edge_index[0].astype(jnp.int32)
    dst = edge_index[1].astype(jnp.int32)
    pad = jnp.full((EPAD - E,), N, jnp.int32)
    src_p = jnp.concatenate([src, pad])
    dst_p = jnp.concatenate([dst, pad])

    src2d = src_p.reshape(EPAD // K, K)
    dst2d = dst_p.reshape(EPAD // K, K)
    deg16 = _deg_kernel(dst2d).reshape(2, NPAD, 16)
    tab1, dis, inv = _tc_prep(deg16, x)
    acc1 = _prop_g2(tab1.reshape(2 * NPAD, 16), src2d, dst2d)
    tab2, hsl = _tc_layer1(acc1.reshape(2, NPAD, 16), x, dis, inv,
                           W1, b1.reshape(1, HID_DIM))
    acc2 = _prop_g4(tab2.reshape(4 * NPAD, 16), src2d, dst2d)
    return _tc_layer2(acc2.reshape(4, NPAD, 16), hsl, dis,
                      W2, b2.reshape(1, OUT_DIM))


# speedup vs baseline: 16.3358x; 1.1076x over previous
import functools

import jax
import jax.numpy as jnp
from jax import lax
from jax.experimental import pallas as pl
from jax.experimental.pallas import tpu as pltpu
from jax.experimental.pallas import tpu_sc as plsc

N = 100000
E = 1600000
IN_DIM, HID_DIM, OUT_DIM = 32, 64, 128

NPAD = 100096
RPT = NPAD // 16
K = 128
EPAD = 1622016
EHALF = EPAD // 2

RB = 256
NBLK = NPAD // RB

_MESH = plsc.VectorSubcoreMesh(core_axis_name="c", subcore_axis_name="s")


def _fill(ref, rows, value):
    def body(i, carry):
        ref[i] = jnp.full((16,), value, jnp.float32)
        return carry
    lax.fori_loop(0, rows, body, 0)


_ZCH = RPT // K + 1


def _zero_acc(acc, zeros, idxv, s):
    def chunk(i, carry):
        base = jnp.minimum(s * RPT + i * K, NPAD - K)

        def zidx(j, carry2):
            idxv[pl.ds(j * 16, 16)] = lax.iota(jnp.int32, 16) + (base + j * 16)
            return carry2

        lax.fori_loop(0, K // 16, zidx, 0)
        pltpu.sync_copy(zeros, acc.at[idxv])
        return carry

    lax.fori_loop(0, _ZCH, chunk, 0)


SUP_D = 12
NSUP_D = 33
SUP_P = 12
NSUP_P = 66


@functools.partial(
    pl.kernel,
    out_type=jax.ShapeDtypeStruct((NPAD, 32), jnp.float32),
    mesh=_MESH,
    compiler_params=pltpu.CompilerParams(use_tc_tiling_on_sc=False),
    scratch_types=[
        pltpu.VMEM((K, 16), jnp.float32),
        pltpu.VMEM((SUP_D, K), jnp.int32),
        pltpu.VMEM_SHARED((NPAD, 16), jnp.float32),
    ],
)
def _deg_kernel(dst2d_hbm, out_hbm, ones, dstb, acc):
    c = lax.axis_index("c")
    s = lax.axis_index("s")
    _fill(ones, K, 0.0)
    _zero_acc(acc, ones, dstb.at[0], s)
    plsc.subcore_barrier()
    _fill(ones, K, 1.0)
    cpt = EHALF // 16 // K

    def body(t, carry):
        rowbase = c * (EHALF // K) + s * cpt + t * SUP_D
        pltpu.sync_copy(dst2d_hbm.at[pl.ds(rowbase, SUP_D)], dstb)
        for u in range(SUP_D):
            pltpu.sync_copy(ones, acc.at[dstb.at[u]], add=True)
        return carry

    lax.fori_loop(0, NSUP_D, body, 0)
    plsc.subcore_barrier()
    pltpu.sync_copy(acc.at[pl.ds(s * RPT, RPT)],
                    out_hbm.at[pl.ds(s * RPT, RPT), pl.ds(c * 16, 16)])


HP = SUP_P // 2
NH = NSUP_P * 2


def _make_prop(G):
    G2 = G // 2
    cpt = EPAD // K // 16

    @functools.partial(
        pl.kernel,
        out_type=jax.ShapeDtypeStruct((NPAD, G * 16), jnp.float32),
        mesh=_MESH,
        compiler_params=pltpu.CompilerParams(use_tc_tiling_on_sc=False),
        scratch_types=[
            pltpu.VMEM((SUP_P, K), jnp.int32),
            pltpu.VMEM((SUP_P, K), jnp.int32),
            pltpu.VMEM((SUP_P, K, 16), jnp.float32),
            pltpu.VMEM_SHARED((NPAD, 16), jnp.float32),
            pltpu.SemaphoreType.DMA,
            pltpu.SemaphoreType.DMA,
        ],
    )
    def prop(table, src2d_hbm, dst2d_hbm, out_hbm,
             gidxb, dstb, rows, acc, sem0, sem1):
        c = lax.axis_index("c")
        s = lax.axis_index("s")

        def load_and_fire(h, sb, g, sem):
            rowbase = s * cpt + h * HP
            pltpu.sync_copy(src2d_hbm.at[pl.ds(rowbase, HP)],
                            gidxb.at[pl.ds(sb, HP)])
            pltpu.sync_copy(dst2d_hbm.at[pl.ds(rowbase, HP)],
                            dstb.at[pl.ds(sb, HP)])

            def off(u, cu):
                def off16(j2, c2):
                    gidxb[sb + u, pl.ds(j2 * 16, 16)] = (
                        gidxb[sb + u, pl.ds(j2 * 16, 16)] * G + g)
                    return c2
                lax.fori_loop(0, K // 16, off16, 0)
                pltpu.async_copy(table.at[gidxb.at[sb + u]],
                                 rows.at[sb + u], sem)
                return cu

            lax.fori_loop(0, HP, off, 0)

        def drain_scatter(sb, sem):
            def drain(u, cu):
                pltpu.make_async_copy(
                    table.at[gidxb.at[sb + u]], rows.at[sb + u], sem).wait()
                return cu

            lax.fori_loop(0, HP, drain, 0)

            def scat(u, cu):
                pltpu.sync_copy(rows.at[sb + u],
                                acc.at[dstb.at[sb + u]], add=True)
                return cu

            lax.fori_loop(0, HP, scat, 0)

        for gi in range(G2):
            g = c * G2 + gi
            _fill(rows.at[0], K, 0.0)
            _zero_acc(acc, rows.at[0], dstb.at[0], s)
            plsc.subcore_barrier()

            load_and_fire(0, 0, g, sem0)

            def body(t, carry):
                load_and_fire(2 * t + 1, HP, g, sem1)
                drain_scatter(0, sem0)

                @pl.when(t < NSUP_P - 1)
                def _():
                    load_and_fire(2 * t + 2, 0, g, sem0)

                drain_scatter(HP, sem1)
                return carry

            lax.fori_loop(0, NSUP_P, body, 0)
            plsc.subcore_barrier()
            pltpu.sync_copy(acc.at[pl.ds(s * RPT, RPT)],
                            out_hbm.at[pl.ds(s * RPT, RPT),
                                       pl.ds(g * 16, 16)])
    return prop


_prop_g2 = _make_prop(2)
_prop_g4 = _make_prop(4)


def _tc_prep(deg16, x):
    def body(deg_ref, x_ref, tab_ref, dis_ref, inv_ref):
        deg = deg_ref[:, 0:1] + deg_ref[:, 16:17] + 1.0
        d = lax.rsqrt(deg)
        tab_ref[...] = x_ref[...] * d
        dis_ref[...] = d
        inv_ref[...] = 1.0 / deg

    return pl.pallas_call(
        body,
        grid=(NBLK,),
        in_specs=[
            pl.BlockSpec((RB, 32), lambda i: (i, 0)),
            pl.BlockSpec((RB, IN_DIM), lambda i: (i, 0)),
        ],
        out_specs=[
            pl.BlockSpec((RB, IN_DIM), lambda i: (i, 0)),
            pl.BlockSpec((RB, 1), lambda i: (i, 0)),
            pl.BlockSpec((RB, 1), lambda i: (i, 0)),
        ],
        out_shape=[
            jax.ShapeDtypeStruct((NPAD, IN_DIM), jnp.float32),
            jax.ShapeDtypeStruct((NPAD, 1), jnp.float32),
            jax.ShapeDtypeStruct((NPAD, 1), jnp.float32),
        ],
    )(deg16, x)


def _tc_layer1(acc1, x, dis, inv, W1, b1):
    def body(acc_ref, x_ref, dis_ref, inv_ref, w_ref, b_ref, tab_ref, hsl_ref):
        d = dis_ref[...]
        iv = inv_ref[...]
        p1 = d * acc_ref[...] + iv * x_ref[...]
        h = jnp.dot(p1, w_ref[...], preferred_element_type=jnp.float32)
        h = jnp.maximum(h + b_ref[...], 0.0)
        tab_ref[...] = h * d
        hsl_ref[...] = h * iv

    return pl.pallas_call(
        body,
        grid=(NBLK,),
        in_specs=[
            pl.BlockSpec((RB, IN_DIM), lambda i: (i, 0)),
            pl.BlockSpec((RB, IN_DIM), lambda i: (i, 0)),
            pl.BlockSpec((RB, 1), lambda i: (i, 0)),
            pl.BlockSpec((RB, 1), lambda i: (i, 0)),
            pl.BlockSpec((IN_DIM, HID_DIM), lambda i: (0, 0)),
            pl.BlockSpec((1, HID_DIM), lambda i: (0, 0)),
        ],
        out_specs=[
            pl.BlockSpec((RB, HID_DIM), lambda i: (i, 0)),
            pl.BlockSpec((RB, HID_DIM), lambda i: (i, 0)),
        ],
        out_shape=[
            jax.ShapeDtypeStruct((NPAD, HID_DIM), jnp.float32),
            jax.ShapeDtypeStruct((NPAD, HID_DIM), jnp.float32),
        ],
    )(acc1, x, dis, inv, W1, b1)


def _tc_layer2(acc2, hsl, dis, W2, b2):
    def body(acc_ref, hsl_ref, dis_ref, w_ref, b_ref, out_ref):
        p2 = dis_ref[...] * acc_ref[...] + hsl_ref[...]
        o = jnp.dot(p2, w_ref[...], preferred_element_type=jnp.float32)
        out_ref[...] = o + b_ref[...]

    return pl.pallas_call(
        body,
        grid=(NBLK,),
        in_specs=[
            pl.BlockSpec((RB, HID_DIM), lambda i: (i, 0)),
            pl.BlockSpec((RB, HID_DIM), lambda i: (i, 0)),
            pl.BlockSpec((RB, 1), lambda i: (i, 0)),
            pl.BlockSpec((HID_DIM, OUT_DIM), lambda i: (0, 0)),
            pl.BlockSpec((1, OUT_DIM), lambda i: (0, 0)),
        ],
        out_specs=pl.BlockSpec((RB, OUT_DIM), lambda i: (i, 0)),
        out_shape=jax.ShapeDtypeStruct((N, OUT_DIM), jnp.float32),
    )(acc2, hsl, dis, W2, b2)


@functools.partial(
    pl.kernel,
    out_type=jax.ShapeDtypeStruct((NPAD, IN_DIM), jnp.float32),
    mesh=_MESH,
    compiler_params=pltpu.CompilerParams(use_tc_tiling_on_sc=False),
    scratch_types=[
        pltpu.VMEM((NPAD // 32,), jnp.int32),
        pltpu.VMEM((NPAD // 32, IN_DIM), jnp.float32),
        pltpu.SemaphoreType.DMA,
    ],
)
def _skel_gather(table_hbm, idx_hbm, out_hbm, idx_v, rows_v, sem):
    bpw = NPAD // 32
    wid = lax.axis_index("s") * 2 + lax.axis_index("c")
    base = wid * bpw
    pltpu.sync_copy(idx_hbm.at[pl.ds(base, bpw)], idx_v)
    pltpu.async_copy(table_hbm.at[idx_v], rows_v, sem).wait()
    pltpu.sync_copy(rows_v, out_hbm.at[pl.ds(base, bpw)])


def kernel(x, edge_index, W1, b1, W2, b2):
    src = edge_index[0].astype(jnp.int32)
    dst = edge_index[1].astype(jnp.int32)
    pad = jnp.full((EPAD - E,), N, jnp.int32)
    src_p = jnp.concatenate([src, pad])
    dst_p = jnp.concatenate([dst, pad])

    src2d = src_p.reshape(EPAD // K, K)
    dst2d = dst_p.reshape(EPAD // K, K)
    deg16 = _deg_kernel(dst2d)
    tab1, dis, inv = _tc_prep(deg16, x)
    acc1 = _prop_g2(tab1.reshape(2 * NPAD, 16), src2d, dst2d)
    tab2, hsl = _tc_layer1(acc1, x, dis, inv, W1, b1.reshape(1, HID_DIM))
    acc2 = _prop_g4(tab2.reshape(4 * NPAD, 16), src2d, dst2d)
    return _tc_layer2(acc2, hsl, dis, W2, b2.reshape(1, OUT_DIM))


# speedup vs baseline: 19.7455x; 1.2087x over previous
import functools

import jax
import jax.numpy as jnp
from jax import lax
from jax.experimental import pallas as pl
from jax.experimental.pallas import tpu as pltpu
from jax.experimental.pallas import tpu_sc as plsc

N = 100000
E = 1600000
IN_DIM, HID_DIM, OUT_DIM = 32, 64, 128

NPAD = 100096
RPT = NPAD // 16
K = 128
EPAD = 1622016
EHALF = EPAD // 2

RB = 1024
NBLK = (NPAD + RB - 1) // RB

_MESH = plsc.VectorSubcoreMesh(core_axis_name="c", subcore_axis_name="s")


def _fill(ref, rows, value):
    def body(i, carry):
        ref[i] = jnp.full((16,), value, jnp.float32)
        return carry
    lax.fori_loop(0, rows, body, 0)


_ZCH = RPT // K + 1


def _zero_acc(acc, zeros, idxv, s):
    def chunk(i, carry):
        base = jnp.minimum(s * RPT + i * K, NPAD - K)

        def zidx(j, carry2):
            idxv[pl.ds(j * 16, 16)] = lax.iota(jnp.int32, 16) + (base + j * 16)
            return carry2

        lax.fori_loop(0, K // 16, zidx, 0)
        pltpu.sync_copy(zeros, acc.at[idxv])
        return carry

    lax.fori_loop(0, _ZCH, chunk, 0)


SUP_D = 12
NSUP_D = 33
SUP_P = 12
NSUP_P = 66


@functools.partial(
    pl.kernel,
    out_type=jax.ShapeDtypeStruct((NPAD, 32), jnp.float32),
    mesh=_MESH,
    compiler_params=pltpu.CompilerParams(use_tc_tiling_on_sc=False),
    scratch_types=[
        pltpu.VMEM((K, 16), jnp.float32),
        pltpu.VMEM((SUP_D, K), jnp.int32),
        pltpu.VMEM_SHARED((NPAD, 16), jnp.float32),
    ],
)
def _deg_kernel(dst2d_hbm, out_hbm, ones, dstb, acc):
    c = lax.axis_index("c")
    s = lax.axis_index("s")
    _fill(ones, K, 0.0)
    _zero_acc(acc, ones, dstb.at[0], s)
    plsc.subcore_barrier()
    _fill(ones, K, 1.0)
    cpt = EHALF // 16 // K

    def body(t, carry):
        rowbase = c * (EHALF // K) + s * cpt + t * SUP_D
        pltpu.sync_copy(dst2d_hbm.at[pl.ds(rowbase, SUP_D)], dstb)
        for u in range(SUP_D):
            pltpu.sync_copy(ones, acc.at[dstb.at[u]], add=True)
        return carry

    lax.fori_loop(0, NSUP_D, body, 0)
    plsc.subcore_barrier()
    pltpu.sync_copy(acc.at[pl.ds(s * RPT, RPT)],
                    out_hbm.at[pl.ds(s * RPT, RPT), pl.ds(c * 16, 16)])


HP = SUP_P // 2
NH = NSUP_P * 2


def _make_prop(G):
    G2 = G // 2
    cpt = EPAD // K // 16

    @functools.partial(
        pl.kernel,
        out_type=jax.ShapeDtypeStruct((NPAD, G * 16), jnp.float32),
        mesh=_MESH,
        compiler_params=pltpu.CompilerParams(use_tc_tiling_on_sc=False),
        scratch_types=[
            pltpu.VMEM((SUP_P, K), jnp.int32),
            pltpu.VMEM((SUP_P, K), jnp.int32),
            pltpu.VMEM((SUP_P, K, 16), jnp.float32),
            pltpu.VMEM_SHARED((NPAD, 16), jnp.float32),
            pltpu.SemaphoreType.DMA,
            pltpu.SemaphoreType.DMA,
        ],
    )
    def prop(table, src2d_hbm, dst2d_hbm, out_hbm,
             gidxb, dstb, rows, acc, sem0, sem1):
        c = lax.axis_index("c")
        s = lax.axis_index("s")

        def load_and_fire(h, sb, g, sem):
            rowbase = s * cpt + h * HP
            pltpu.sync_copy(src2d_hbm.at[pl.ds(rowbase, HP)],
                            gidxb.at[pl.ds(sb, HP)])
            pltpu.sync_copy(dst2d_hbm.at[pl.ds(rowbase, HP)],
                            dstb.at[pl.ds(sb, HP)])

            def off(u, cu):
                def off16(j2, c2):
                    gidxb[sb + u, pl.ds(j2 * 16, 16)] = (
                        gidxb[sb + u, pl.ds(j2 * 16, 16)] * G + g)
                    return c2
                lax.fori_loop(0, K // 16, off16, 0)
                pltpu.async_copy(table.at[gidxb.at[sb + u]],
                                 rows.at[sb + u], sem)
                return cu

            lax.fori_loop(0, HP, off, 0)

        def drain_scatter(sb, sem):
            def drain(u, cu):
                pltpu.make_async_copy(
                    table.at[gidxb.at[sb + u]], rows.at[sb + u], sem).wait()
                return cu

            lax.fori_loop(0, HP, drain, 0)

            def scat(u, cu):
                pltpu.sync_copy(rows.at[sb + u],
                                acc.at[dstb.at[sb + u]], add=True)
                return cu

            lax.fori_loop(0, HP, scat, 0)

        for gi in range(G2):
            g = c * G2 + gi
            _fill(rows.at[0], K, 0.0)
            _zero_acc(acc, rows.at[0], dstb.at[0], s)
            plsc.subcore_barrier()

            load_and_fire(0, 0, g, sem0)

            def body(t, carry):
                load_and_fire(2 * t + 1, HP, g, sem1)
                drain_scatter(0, sem0)

                @pl.when(t < NSUP_P - 1)
                def _():
                    load_and_fire(2 * t + 2, 0, g, sem0)

                drain_scatter(HP, sem1)
                return carry

            lax.fori_loop(0, NSUP_P, body, 0)
            plsc.subcore_barrier()
            pltpu.sync_copy(acc.at[pl.ds(s * RPT, RPT)],
                            out_hbm.at[pl.ds(s * RPT, RPT),
                                       pl.ds(g * 16, 16)])
    return prop


_prop_g2 = _make_prop(2)
_prop_g4 = _make_prop(4)


def _tc_prep(deg16, x):
    def body(deg_ref, x_ref, tab_ref, dis_ref, inv_ref):
        deg = deg_ref[:, 0:1] + deg_ref[:, 16:17] + 1.0
        d = lax.rsqrt(deg)
        tab_ref[...] = x_ref[...] * d
        dis_ref[...] = d
        inv_ref[...] = 1.0 / deg

    return pl.pallas_call(
        body,
        grid=(NBLK,),
        in_specs=[
            pl.BlockSpec((RB, 32), lambda i: (i, 0)),
            pl.BlockSpec((RB, IN_DIM), lambda i: (i, 0)),
        ],
        out_specs=[
            pl.BlockSpec((RB, IN_DIM), lambda i: (i, 0)),
            pl.BlockSpec((RB, 1), lambda i: (i, 0)),
            pl.BlockSpec((RB, 1), lambda i: (i, 0)),
        ],
        out_shape=[
            jax.ShapeDtypeStruct((NPAD, IN_DIM), jnp.float32),
            jax.ShapeDtypeStruct((NPAD, 1), jnp.float32),
            jax.ShapeDtypeStruct((NPAD, 1), jnp.float32),
        ],
    )(deg16, x)


def _tc_layer1(acc1, x, dis, inv, W1, b1):
    def body(acc_ref, x_ref, dis_ref, inv_ref, w_ref, b_ref, tab_ref, hsl_ref):
        d = dis_ref[...]
        iv = inv_ref[...]
        p1 = d * acc_ref[...] + iv * x_ref[...]
        h = jnp.dot(p1, w_ref[...], preferred_element_type=jnp.float32)
        h = jnp.maximum(h + b_ref[...], 0.0)
        tab_ref[...] = h * d
        hsl_ref[...] = h * iv

    return pl.pallas_call(
        body,
        grid=(NBLK,),
        in_specs=[
            pl.BlockSpec((RB, IN_DIM), lambda i: (i, 0)),
            pl.BlockSpec((RB, IN_DIM), lambda i: (i, 0)),
            pl.BlockSpec((RB, 1), lambda i: (i, 0)),
            pl.BlockSpec((RB, 1), lambda i: (i, 0)),
            pl.BlockSpec((IN_DIM, HID_DIM), lambda i: (0, 0)),
            pl.BlockSpec((1, HID_DIM), lambda i: (0, 0)),
        ],
        out_specs=[
            pl.BlockSpec((RB, HID_DIM), lambda i: (i, 0)),
            pl.BlockSpec((RB, HID_DIM), lambda i: (i, 0)),
        ],
        out_shape=[
            jax.ShapeDtypeStruct((NPAD, HID_DIM), jnp.float32),
            jax.ShapeDtypeStruct((NPAD, HID_DIM), jnp.float32),
        ],
    )(acc1, x, dis, inv, W1, b1)


def _tc_layer2(acc2, hsl, dis, W2, b2):
    def body(acc_ref, hsl_ref, dis_ref, w_ref, b_ref, out_ref):
        p2 = dis_ref[...] * acc_ref[...] + hsl_ref[...]
        o = jnp.dot(p2, w_ref[...], preferred_element_type=jnp.float32)
        out_ref[...] = o + b_ref[...]

    return pl.pallas_call(
        body,
        grid=(NBLK,),
        in_specs=[
            pl.BlockSpec((RB, HID_DIM), lambda i: (i, 0)),
            pl.BlockSpec((RB, HID_DIM), lambda i: (i, 0)),
            pl.BlockSpec((RB, 1), lambda i: (i, 0)),
            pl.BlockSpec((HID_DIM, OUT_DIM), lambda i: (0, 0)),
            pl.BlockSpec((1, OUT_DIM), lambda i: (0, 0)),
        ],
        out_specs=pl.BlockSpec((RB, OUT_DIM), lambda i: (i, 0)),
        out_shape=jax.ShapeDtypeStruct((N, OUT_DIM), jnp.float32),
    )(acc2, hsl, dis, W2, b2)


@functools.partial(
    pl.kernel,
    out_type=jax.ShapeDtypeStruct((NPAD, IN_DIM), jnp.float32),
    mesh=_MESH,
    compiler_params=pltpu.CompilerParams(use_tc_tiling_on_sc=False),
    scratch_types=[
        pltpu.VMEM((NPAD // 32,), jnp.int32),
        pltpu.VMEM((NPAD // 32, IN_DIM), jnp.float32),
        pltpu.SemaphoreType.DMA,
    ],
)
def _skel_gather(table_hbm, idx_hbm, out_hbm, idx_v, rows_v, sem):
    bpw = NPAD // 32
    wid = lax.axis_index("s") * 2 + lax.axis_index("c")
    base = wid * bpw
    pltpu.sync_copy(idx_hbm.at[pl.ds(base, bpw)], idx_v)
    pltpu.async_copy(table_hbm.at[idx_v], rows_v, sem).wait()
    pltpu.sync_copy(rows_v, out_hbm.at[pl.ds(base, bpw)])


def kernel(x, edge_index, W1, b1, W2, b2):
    src = edge_index[0].astype(jnp.int32)
    dst = edge_index[1].astype(jnp.int32)
    pad = jnp.full((EPAD - E,), N, jnp.int32)
    src_p = jnp.concatenate([src, pad])
    dst_p = jnp.concatenate([dst, pad])

    src2d = src_p.reshape(EPAD // K, K)
    dst2d = dst_p.reshape(EPAD // K, K)
    deg16 = _deg_kernel(dst2d)
    tab1, dis, inv = _tc_prep(deg16, x)
    acc1 = _prop_g2(tab1.reshape(2 * NPAD, 16), src2d, dst2d)
    tab2, hsl = _tc_layer1(acc1, x, dis, inv, W1, b1.reshape(1, HID_DIM))
    acc2 = _prop_g4(tab2.reshape(4 * NPAD, 16), src2d, dst2d)
    return _tc_layer2(acc2, hsl, dis, W2, b2.reshape(1, OUT_DIM))


# speedup vs baseline: 27.0600x; 1.3704x over previous
import functools

import jax
import jax.numpy as jnp
from jax import lax
from jax.experimental import pallas as pl
from jax.experimental.pallas import tpu as pltpu
from jax.experimental.pallas import tpu_sc as plsc

N = 100000
E = 1600000
IN_DIM, HID_DIM, OUT_DIM = 32, 64, 128

NPAD = 100096
RPT = NPAD // 16
K = 128
EPAD = 1622016
EHALF = EPAD // 2

RB = 1024
NBLK = (NPAD + RB - 1) // RB

_MESH = plsc.VectorSubcoreMesh(core_axis_name="c", subcore_axis_name="s")


def _fill(ref, rows, value):
    def body(i, carry):
        ref[i] = jnp.full((16,), value, jnp.float32)
        return carry
    lax.fori_loop(0, rows, body, 0)


_ZCH = RPT // K + 1


def _zero_acc(acc, zeros, idxv, s):
    def chunk(i, carry):
        base = jnp.minimum(s * RPT + i * K, NPAD - K)

        def zidx(j, carry2):
            idxv[pl.ds(j * 16, 16)] = lax.iota(jnp.int32, 16) + (base + j * 16)
            return carry2

        lax.fori_loop(0, K // 16, zidx, 0)
        pltpu.sync_copy(zeros, acc.at[idxv])
        return carry

    lax.fori_loop(0, _ZCH, chunk, 0)


SUP_D = 12
NSUP_D = 33
SUP_P = 12
NSUP_P = 66


@functools.partial(
    pl.kernel,
    out_type=jax.ShapeDtypeStruct((NPAD, 32), jnp.float32),
    mesh=_MESH,
    compiler_params=pltpu.CompilerParams(use_tc_tiling_on_sc=False),
    scratch_types=[
        pltpu.VMEM((K, 16), jnp.float32),
        pltpu.VMEM((SUP_D, K), jnp.int32),
        pltpu.VMEM_SHARED((NPAD, 16), jnp.float32),
    ],
)
def _deg_kernel(dst2d_hbm, out_hbm, ones, dstb, acc):
    c = lax.axis_index("c")
    s = lax.axis_index("s")
    _fill(ones, K, 0.0)
    _zero_acc(acc, ones, dstb.at[0], s)
    plsc.subcore_barrier()
    _fill(ones, K, 1.0)
    cpt = EHALF // 16 // K

    def body(t, carry):
        rowbase = c * (EHALF // K) + s * cpt + t * SUP_D
        pltpu.sync_copy(dst2d_hbm.at[pl.ds(rowbase, SUP_D)], dstb)
        for u in range(SUP_D):
            pltpu.sync_copy(ones, acc.at[dstb.at[u]], add=True)
        return carry

    lax.fori_loop(0, NSUP_D, body, 0)
    plsc.subcore_barrier()
    pltpu.sync_copy(acc.at[pl.ds(s * RPT, RPT)],
                    out_hbm.at[pl.ds(s * RPT, RPT), pl.ds(c * 16, 16)])


HP = SUP_P // 2
NH = NSUP_P * 2


def _fill_bf16(ref, rows, value):
    def body(i, carry):
        ref[i] = jnp.full((32,), value, jnp.bfloat16)
        return carry
    lax.fori_loop(0, rows, body, 0)


def _make_prop(split_edges):
    cpt = EPAD // K // 16 // (2 if split_edges else 1)
    nh = cpt // HP
    nsup = nh // 2
    out_ty = (jax.ShapeDtypeStruct((2 * NPAD, 32), jnp.bfloat16)
              if split_edges
              else jax.ShapeDtypeStruct((NPAD, 64), jnp.bfloat16))

    @functools.partial(
        pl.kernel,
        out_type=out_ty,
        mesh=_MESH,
        compiler_params=pltpu.CompilerParams(use_tc_tiling_on_sc=False),
        scratch_types=[
            pltpu.VMEM((SUP_P, K), jnp.int32),
            pltpu.VMEM((SUP_P, K), jnp.int32),
            pltpu.VMEM((SUP_P, K, 32), jnp.bfloat16),
            pltpu.VMEM_SHARED((NPAD, 32), jnp.bfloat16),
            pltpu.SemaphoreType.DMA,
            pltpu.SemaphoreType.DMA,
        ],
    )
    def prop(table, src2d_hbm, dst2d_hbm, out_hbm,
             gidxb, dstb, rows, acc, sem0, sem1):
        c = lax.axis_index("c")
        s = lax.axis_index("s")
        edgebase = c * (EPAD // K // 2) if split_edges else 0

        def load_and_fire(h, sb, sem):
            rowbase = edgebase + s * cpt + h * HP
            pltpu.sync_copy(src2d_hbm.at[pl.ds(rowbase, HP)],
                            gidxb.at[pl.ds(sb, HP)])
            pltpu.sync_copy(dst2d_hbm.at[pl.ds(rowbase, HP)],
                            dstb.at[pl.ds(sb, HP)])

            def off(u, cu):
                if not split_edges:
                    def off16(j2, c2):
                        gidxb[sb + u, pl.ds(j2 * 16, 16)] = (
                            gidxb[sb + u, pl.ds(j2 * 16, 16)] * 2 + c)
                        return c2
                    lax.fori_loop(0, K // 16, off16, 0)
                pltpu.async_copy(table.at[gidxb.at[sb + u]],
                                 rows.at[sb + u], sem)
                return cu

            lax.fori_loop(0, HP, off, 0)

        def drain_scatter(sb, sem):
            def drain(u, cu):
                pltpu.make_async_copy(
                    table.at[gidxb.at[sb + u]], rows.at[sb + u], sem).wait()
                return cu

            lax.fori_loop(0, HP, drain, 0)

            def scat(u, cu):
                pltpu.sync_copy(rows.at[sb + u],
                                acc.at[dstb.at[sb + u]], add=True)
                return cu

            lax.fori_loop(0, HP, scat, 0)

        _fill_bf16(rows.at[0], K, 0.0)
        _zero_acc(acc, rows.at[0], dstb.at[0], s)
        plsc.subcore_barrier()

        load_and_fire(0, 0, sem0)

        def body(t, carry):
            load_and_fire(2 * t + 1, HP, sem1)
            drain_scatter(0, sem0)

            @pl.when(t < nsup - 1)
            def _():
                load_and_fire(2 * t + 2, 0, sem0)

            drain_scatter(HP, sem1)
            return carry

        lax.fori_loop(0, nsup, body, 0)
        plsc.subcore_barrier()
        if split_edges:
            pltpu.sync_copy(acc.at[pl.ds(s * RPT, RPT)],
                            out_hbm.at[pl.ds(c * NPAD + s * RPT, RPT)])
        else:
            pltpu.sync_copy(acc.at[pl.ds(s * RPT, RPT)],
                            out_hbm.at[pl.ds(s * RPT, RPT),
                                       pl.ds(c * 32, 32)])
    return prop


_prop_l1 = _make_prop(True)
_prop_l2 = _make_prop(False)


def _tc_prep(deg16, x):
    def body(deg_ref, x_ref, tab_ref, dis_ref, inv_ref):
        deg = deg_ref[:, 0:1] + deg_ref[:, 16:17] + 1.0
        d = lax.rsqrt(deg)
        tab_ref[...] = (x_ref[...] * d).astype(jnp.bfloat16)
        dis_ref[...] = d
        inv_ref[...] = 1.0 / deg

    return pl.pallas_call(
        body,
        grid=(NBLK,),
        in_specs=[
            pl.BlockSpec((RB, 32), lambda i: (i, 0)),
            pl.BlockSpec((RB, IN_DIM), lambda i: (i, 0)),
        ],
        out_specs=[
            pl.BlockSpec((RB, IN_DIM), lambda i: (i, 0)),
            pl.BlockSpec((RB, 1), lambda i: (i, 0)),
            pl.BlockSpec((RB, 1), lambda i: (i, 0)),
        ],
        out_shape=[
            jax.ShapeDtypeStruct((NPAD, IN_DIM), jnp.bfloat16),
            jax.ShapeDtypeStruct((NPAD, 1), jnp.float32),
            jax.ShapeDtypeStruct((NPAD, 1), jnp.float32),
        ],
    )(deg16, x)


def _tc_layer1(acc1, x, dis, inv, W1, b1):
    def body(acc_ref, x_ref, dis_ref, inv_ref, w_ref, b_ref, tab_ref, hsl_ref):
        d = dis_ref[...]
        iv = inv_ref[...]
        accs = (acc_ref[0].astype(jnp.float32)
                + acc_ref[1].astype(jnp.float32))
        p1 = d * accs + iv * x_ref[...]
        h = jnp.dot(p1, w_ref[...], preferred_element_type=jnp.float32)
        h = jnp.maximum(h + b_ref[...], 0.0)
        tab_ref[...] = (h * d).astype(jnp.bfloat16)
        hsl_ref[...] = h * iv

    return pl.pallas_call(
        body,
        grid=(NBLK,),
        in_specs=[
            pl.BlockSpec((2, RB, IN_DIM), lambda i: (0, i, 0)),
            pl.BlockSpec((RB, IN_DIM), lambda i: (i, 0)),
            pl.BlockSpec((RB, 1), lambda i: (i, 0)),
            pl.BlockSpec((RB, 1), lambda i: (i, 0)),
            pl.BlockSpec((IN_DIM, HID_DIM), lambda i: (0, 0)),
            pl.BlockSpec((1, HID_DIM), lambda i: (0, 0)),
        ],
        out_specs=[
            pl.BlockSpec((RB, HID_DIM), lambda i: (i, 0)),
            pl.BlockSpec((RB, HID_DIM), lambda i: (i, 0)),
        ],
        out_shape=[
            jax.ShapeDtypeStruct((NPAD, HID_DIM), jnp.bfloat16),
            jax.ShapeDtypeStruct((NPAD, HID_DIM), jnp.float32),
        ],
    )(acc1, x, dis, inv, W1, b1)


def _tc_layer2(acc2, hsl, dis, W2, b2):
    def body(acc_ref, hsl_ref, dis_ref, w_ref, b_ref, out_ref):
        p2 = (dis_ref[...] * acc_ref[...].astype(jnp.float32)
              + hsl_ref[...])
        o = jnp.dot(p2, w_ref[...], preferred_element_type=jnp.float32)
        out_ref[...] = o + b_ref[...]

    return pl.pallas_call(
        body,
        grid=(NBLK,),
        in_specs=[
            pl.BlockSpec((RB, HID_DIM), lambda i: (i, 0)),
            pl.BlockSpec((RB, HID_DIM), lambda i: (i, 0)),
            pl.BlockSpec((RB, 1), lambda i: (i, 0)),
            pl.BlockSpec((HID_DIM, OUT_DIM), lambda i: (0, 0)),
            pl.BlockSpec((1, OUT_DIM), lambda i: (0, 0)),
        ],
        out_specs=pl.BlockSpec((RB, OUT_DIM), lambda i: (i, 0)),
        out_shape=jax.ShapeDtypeStruct((N, OUT_DIM), jnp.float32),
    )(acc2, hsl, dis, W2, b2)


@functools.partial(
    pl.kernel,
    out_type=jax.ShapeDtypeStruct((NPAD, IN_DIM), jnp.float32),
    mesh=_MESH,
    compiler_params=pltpu.CompilerParams(use_tc_tiling_on_sc=False),
    scratch_types=[
        pltpu.VMEM((NPAD // 32,), jnp.int32),
        pltpu.VMEM((NPAD // 32, IN_DIM), jnp.float32),
        pltpu.SemaphoreType.DMA,
    ],
)
def _skel_gather(table_hbm, idx_hbm, out_hbm, idx_v, rows_v, sem):
    bpw = NPAD // 32
    wid = lax.axis_index("s") * 2 + lax.axis_index("c")
    base = wid * bpw
    pltpu.sync_copy(idx_hbm.at[pl.ds(base, bpw)], idx_v)
    pltpu.async_copy(table_hbm.at[idx_v], rows_v, sem).wait()
    pltpu.sync_copy(rows_v, out_hbm.at[pl.ds(base, bpw)])


def kernel(x, edge_index, W1, b1, W2, b2):
    src = edge_index[0].astype(jnp.int32)
    dst = edge_index[1].astype(jnp.int32)
    pad = jnp.full((EPAD - E,), N, jnp.int32)
    src_p = jnp.concatenate([src, pad])
    dst_p = jnp.concatenate([dst, pad])

    src2d = src_p.reshape(EPAD // K, K)
    dst2d = dst_p.reshape(EPAD // K, K)
    deg16 = _deg_kernel(dst2d)
    tab1, dis, inv = _tc_prep(deg16, x)
    acc1 = _prop_l1(tab1, src2d, dst2d)
    tab2, hsl = _tc_layer1(acc1.reshape(2, NPAD, IN_DIM), x, dis, inv,
                           W1, b1.reshape(1, HID_DIM))
    acc2 = _prop_l2(tab2.reshape(2 * NPAD, 32), src2d, dst2d)
    return _tc_layer2(acc2, hsl, dis, W2, b2.reshape(1, OUT_DIM))


# speedup vs baseline: 27.0978x; 1.0014x over previous
import functools

import jax
import jax.numpy as jnp
from jax import lax
from jax.experimental import pallas as pl
from jax.experimental.pallas import tpu as pltpu
from jax.experimental.pallas import tpu_sc as plsc

N = 100000
E = 1600000
IN_DIM, HID_DIM, OUT_DIM = 32, 64, 128

NPAD = 100096
RPT = NPAD // 16
K = 128
EPAD = 1622016
EHALF = EPAD // 2

RB = 1024
NBLK = (NPAD + RB - 1) // RB

_MESH = plsc.VectorSubcoreMesh(core_axis_name="c", subcore_axis_name="s")


def _fill(ref, rows, value):
    def body(i, carry):
        ref[i] = jnp.full((16,), value, jnp.float32)
        return carry
    lax.fori_loop(0, rows, body, 0)


_ZCH = RPT // K + 1


def _zero_acc(acc, zeros, idxv, s):
    def chunk(i, carry):
        base = jnp.minimum(s * RPT + i * K, NPAD - K)

        def zidx(j, carry2):
            idxv[pl.ds(j * 16, 16)] = lax.iota(jnp.int32, 16) + (base + j * 16)
            return carry2

        lax.fori_loop(0, K // 16, zidx, 0)
        pltpu.sync_copy(zeros, acc.at[idxv])
        return carry

    lax.fori_loop(0, _ZCH, chunk, 0)


SUP_D = 12
NSUP_D = 33
SUP_P = 12
NSUP_P = 66


@functools.partial(
    pl.kernel,
    out_type=jax.ShapeDtypeStruct((NPAD, 32), jnp.float32),
    mesh=_MESH,
    compiler_params=pltpu.CompilerParams(use_tc_tiling_on_sc=False),
    scratch_types=[
        pltpu.VMEM((K, 16), jnp.float32),
        pltpu.VMEM((SUP_D, K), jnp.int32),
        pltpu.VMEM_SHARED((NPAD, 16), jnp.float32),
    ],
)
def _deg_kernel(dst2d_hbm, out_hbm, ones, dstb, acc):
    c = lax.axis_index("c")
    s = lax.axis_index("s")
    _fill(ones, K, 0.0)
    _zero_acc(acc, ones, dstb.at[0], s)
    plsc.subcore_barrier()
    _fill(ones, K, 1.0)
    cpt = EHALF // 16 // K

    def body(t, carry):
        rowbase = c * (EHALF // K) + s * cpt + t * SUP_D
        pltpu.sync_copy(dst2d_hbm.at[pl.ds(rowbase, SUP_D)], dstb)
        for u in range(SUP_D):
            pltpu.sync_copy(ones, acc.at[dstb.at[u]], add=True)
        return carry

    lax.fori_loop(0, NSUP_D, body, 0)
    plsc.subcore_barrier()
    pltpu.sync_copy(acc.at[pl.ds(s * RPT, RPT)],
                    out_hbm.at[pl.ds(s * RPT, RPT), pl.ds(c * 16, 16)])


HP = SUP_P // 2
NH = NSUP_P * 2


def _fill_bf16(ref, rows, value):
    def body(i, carry):
        ref[i] = jnp.full((32,), value, jnp.bfloat16)
        return carry
    lax.fori_loop(0, rows, body, 0)


def _make_prop(split_edges):
    cpt = EPAD // K // 16 // (2 if split_edges else 1)
    nh = cpt // HP
    nsup = nh // 2
    out_ty = (jax.ShapeDtypeStruct((2 * NPAD, 32), jnp.bfloat16)
              if split_edges
              else jax.ShapeDtypeStruct((NPAD, 64), jnp.bfloat16))

    @functools.partial(
        pl.kernel,
        out_type=out_ty,
        mesh=_MESH,
        compiler_params=pltpu.CompilerParams(use_tc_tiling_on_sc=False),
        scratch_types=[
            pltpu.VMEM((SUP_P, K), jnp.int32),
            pltpu.VMEM((SUP_P, K), jnp.int32),
            pltpu.VMEM((SUP_P, K, 32), jnp.bfloat16),
            pltpu.VMEM_SHARED((NPAD, 32), jnp.bfloat16),
            pltpu.SemaphoreType.DMA,
            pltpu.SemaphoreType.DMA,
        ],
    )
    def prop(table, src2d_hbm, dst2d_hbm, out_hbm,
             gidxb, dstb, rows, acc, sem0, sem1):
        c = lax.axis_index("c")
        s = lax.axis_index("s")
        edgebase = c * (EPAD // K // 2) if split_edges else 0

        def load_and_fire(h, sb, sem):
            rowbase = edgebase + s * cpt + h * HP
            pltpu.sync_copy(src2d_hbm.at[pl.ds(rowbase, HP)],
                            gidxb.at[pl.ds(sb, HP)])
            pltpu.sync_copy(dst2d_hbm.at[pl.ds(rowbase, HP)],
                            dstb.at[pl.ds(sb, HP)])

            def off(u, cu):
                if not split_edges:
                    def off16(j2, c2):
                        gidxb[sb + u, pl.ds(j2 * 16, 16)] = (
                            gidxb[sb + u, pl.ds(j2 * 16, 16)] * 2 + c)
                        return c2
                    lax.fori_loop(0, K // 16, off16, 0)
                pltpu.async_copy(table.at[gidxb.at[sb + u]],
                                 rows.at[sb + u], sem)
                return cu

            lax.fori_loop(0, HP, off, 0)

        def drain_scatter(sb, sem):
            def drain(u, cu):
                pltpu.make_async_copy(
                    table.at[gidxb.at[sb + u]], rows.at[sb + u], sem).wait()
                return cu

            lax.fori_loop(0, HP, drain, 0)

            def scat(u, cu):
                pltpu.sync_copy(rows.at[sb + u],
                                acc.at[dstb.at[sb + u]], add=True)
                return cu

            lax.fori_loop(0, HP, scat, 0)

        _fill_bf16(rows.at[0], K, 0.0)
        _zero_acc(acc, rows.at[0], dstb.at[0], s)
        plsc.subcore_barrier()

        load_and_fire(0, 0, sem0)

        def body(t, carry):
            load_and_fire(2 * t + 1, HP, sem1)
            drain_scatter(0, sem0)

            @pl.when(t < nsup - 1)
            def _():
                load_and_fire(2 * t + 2, 0, sem0)

            drain_scatter(HP, sem1)
            return carry

        lax.fori_loop(0, nsup, body, 0)
        plsc.subcore_barrier()
        if split_edges:
            pltpu.sync_copy(acc.at[pl.ds(s * RPT, RPT)],
                            out_hbm.at[pl.ds(c * NPAD + s * RPT, RPT)])
        else:
            pltpu.sync_copy(acc.at[pl.ds(s * RPT, RPT)],
                            out_hbm.at[pl.ds(s * RPT, RPT),
                                       pl.ds(c * 32, 32)])
    return prop


_prop_l1 = _make_prop(True)
_prop_l2 = _make_prop(False)


def _tc_prep(deg16, x):
    def body(deg_ref, x_ref, tab_ref, dis_ref, inv_ref):
        deg = deg_ref[:, 0:1] + deg_ref[:, 16:17] + 1.0
        d = lax.rsqrt(deg)
        tab_ref[...] = (x_ref[...] * d).astype(jnp.bfloat16)
        dis_ref[...] = d
        inv_ref[...] = 1.0 / deg

    return pl.pallas_call(
        body,
        grid=(NBLK,),
        in_specs=[
            pl.BlockSpec((RB, 32), lambda i: (i, 0)),
            pl.BlockSpec((RB, IN_DIM), lambda i: (i, 0)),
        ],
        out_specs=[
            pl.BlockSpec((RB, IN_DIM), lambda i: (i, 0)),
            pl.BlockSpec((RB, 1), lambda i: (i, 0)),
            pl.BlockSpec((RB, 1), lambda i: (i, 0)),
        ],
        out_shape=[
            jax.ShapeDtypeStruct((NPAD, IN_DIM), jnp.bfloat16),
            jax.ShapeDtypeStruct((NPAD, 1), jnp.float32),
            jax.ShapeDtypeStruct((NPAD, 1), jnp.float32),
        ],
    )(deg16, x)


def _tc_layer1(acc1, x, dis, inv, W1, b1):
    def body(acc_ref, x_ref, dis_ref, inv_ref, w_ref, b_ref, tab_ref, hsl_ref):
        d = dis_ref[...]
        iv = inv_ref[...]
        accs = (acc_ref[0].astype(jnp.float32)
                + acc_ref[1].astype(jnp.float32))
        p1 = d * accs + iv * x_ref[...]
        h = jnp.dot(p1, w_ref[...], preferred_element_type=jnp.float32)
        h = jnp.maximum(h + b_ref[...], 0.0)
        tab_ref[...] = (h * d).astype(jnp.bfloat16)
        hsl_ref[...] = h * iv

    return pl.pallas_call(
        body,
        grid=(NBLK,),
        in_specs=[
            pl.BlockSpec((2, RB, IN_DIM), lambda i: (0, i, 0)),
            pl.BlockSpec((RB, IN_DIM), lambda i: (i, 0)),
            pl.BlockSpec((RB, 1), lambda i: (i, 0)),
            pl.BlockSpec((RB, 1), lambda i: (i, 0)),
            pl.BlockSpec((IN_DIM, HID_DIM), lambda i: (0, 0)),
            pl.BlockSpec((1, HID_DIM), lambda i: (0, 0)),
        ],
        out_specs=[
            pl.BlockSpec((RB, HID_DIM), lambda i: (i, 0)),
            pl.BlockSpec((RB, HID_DIM), lambda i: (i, 0)),
        ],
        out_shape=[
            jax.ShapeDtypeStruct((NPAD, HID_DIM), jnp.bfloat16),
            jax.ShapeDtypeStruct((NPAD, HID_DIM), jnp.float32),
        ],
    )(acc1, x, dis, inv, W1, b1)


def _tc_layer2(acc2, hsl, dis, W2, b2):
    def body(acc_ref, hsl_ref, dis_ref, w_ref, b_ref, out_ref):
        p2 = (dis_ref[...] * acc_ref[...].astype(jnp.float32)
              + hsl_ref[...])
        o = jnp.dot(p2, w_ref[...], preferred_element_type=jnp.float32)
        out_ref[...] = o + b_ref[...]

    return pl.pallas_call(
        body,
        grid=(NBLK,),
        in_specs=[
            pl.BlockSpec((RB, HID_DIM), lambda i: (i, 0)),
            pl.BlockSpec((RB, HID_DIM), lambda i: (i, 0)),
            pl.BlockSpec((RB, 1), lambda i: (i, 0)),
            pl.BlockSpec((HID_DIM, OUT_DIM), lambda i: (0, 0)),
            pl.BlockSpec((1, OUT_DIM), lambda i: (0, 0)),
        ],
        out_specs=pl.BlockSpec((RB, OUT_DIM), lambda i: (i, 0)),
        out_shape=jax.ShapeDtypeStruct((N, OUT_DIM), jnp.float32),
    )(acc2, hsl, dis, W2, b2)


def kernel(x, edge_index, W1, b1, W2, b2):
    src = edge_index[0].astype(jnp.int32)
    dst = edge_index[1].astype(jnp.int32)
    pad = jnp.full((EPAD - E,), N, jnp.int32)
    src_p = jnp.concatenate([src, pad])
    dst_p = jnp.concatenate([dst, pad])

    src2d = src_p.reshape(EPAD // K, K)
    dst2d = dst_p.reshape(EPAD // K, K)
    deg16 = _deg_kernel(dst2d)
    tab1, dis, inv = _tc_prep(deg16, x)
    acc1 = _prop_l1(tab1, src2d, dst2d)
    tab2, hsl = _tc_layer1(acc1.reshape(2, NPAD, IN_DIM), x, dis, inv,
                           W1, b1.reshape(1, HID_DIM))
    acc2 = _prop_l2(tab2.reshape(2 * NPAD, 32), src2d, dst2d)
    return _tc_layer2(acc2, hsl, dis, W2, b2.reshape(1, OUT_DIM))
